# baseline (speedup 1.0000x reference)
import functools

import jax
import jax.numpy as jnp
from jax import lax
from jax.experimental import pallas as pl
from jax.experimental.pallas import tpu as pltpu
from jax.experimental.pallas import tpu_sc as plsc

_B, _C, _HW = 1024, 10, 4096
_NC, _NS = 2, 16
_NW = _NC * _NS
_NBW = _B // _NW
_CH = _C // 2
_CW = _CH * _HW
_L = 16


def _build_fill_call():
    mesh = plsc.VectorSubcoreMesh(core_axis_name="c", subcore_axis_name="s")

    @functools.partial(
        pl.kernel,
        out_type=jax.ShapeDtypeStruct((_B * 2, _CW), jnp.float32),
        mesh=mesh,
        scratch_types=[
            pltpu.VMEM((_CW,), jnp.float32),
            pltpu.VMEM((_CW,), jnp.float32),
            pltpu.VMEM((_CW,), jnp.float32),
            pltpu.VMEM((_CW,), jnp.float32),
            pltpu.VMEM((_HW,), jnp.float32),
            pltpu.VMEM((_HW,), jnp.float32),
            pltpu.VMEM((_C * _L,), jnp.float32),
            pltpu.VMEM((_L,), jnp.float32),
            pltpu.SemaphoreType.DMA,
            pltpu.SemaphoreType.DMA,
            pltpu.SemaphoreType.DMA,
            pltpu.SemaphoreType.DMA,
            pltpu.SemaphoreType.DMA,
            pltpu.SemaphoreType.DMA,
        ],
    )
    def fill_kernel(grid_h, mask_h, fill_h, thr_h, out_h,
                    gin0, gin1, gout0, gout1, m0, m1, fvm, tvm,
                    sg0, sg1, sm0, sm1, so0, so1):
        wid = lax.axis_index("s") * _NC + lax.axis_index("c")
        b0 = wid * _NBW

        pltpu.sync_copy(fill_h, fvm)
        pltpu.sync_copy(thr_h, tvm)
        thrv = tvm[...]
        fills = [fvm[pl.ds(c * _L, _L)] for c in range(_C)]

        gins, gouts, ms = (gin0, gin1), (gout0, gout1), (m0, m1)
        sgs, sms, sos = (sg0, sg1), (sm0, sm1), (so0, so1)

        def in_descs(g, s):
            b = b0 + g
            return (
                pltpu.make_async_copy(
                    grid_h.at[2 * b + s], gins[s], sgs[s]),
                pltpu.make_async_copy(
                    mask_h.at[b], ms[s], sms[s]),
            )

        def out_desc(g, s):
            b = b0 + g
            return pltpu.make_async_copy(
                gouts[s], out_h.at[2 * b + s], sos[s])

        def compute(s):
            gi, go, mb = gins[s], gouts[s], ms[s]
            cfills = fills[s * _CH:(s + 1) * _CH]

            def jbody(j, carry):
                o = j * _L
                pred = mb[pl.ds(o, _L)] > thrv
                for cl in range(_CH):
                    gv = gi[pl.ds(cl * _HW + o, _L)]
                    go[pl.ds(cl * _HW + o, _L)] = jnp.where(
                        pred, cfills[cl], gv)
                return carry

            lax.fori_loop(0, _HW // _L, jbody, 0, unroll=2)

        for d in in_descs(0, 0):
            d.start()
        for d in in_descs(0, 1):
            d.start()

        def gbody(g, carry):
            for s in (0, 1):
                for d in in_descs(g, s):
                    d.wait()

                @pl.when(g >= 1)
                def _wait_prev_out():
                    out_desc(g - 1, s).wait()

                compute(s)
                out_desc(g, s).start()

                @pl.when(g <= _NBW - 2)
                def _start_next_in():
                    for d in in_descs(g + 1, s):
                        d.start()
            return carry

        lax.fori_loop(0, _NBW, gbody, 0)
        out_desc(_NBW - 1, 0).wait()
        out_desc(_NBW - 1, 1).wait()

    return fill_kernel


_sc_fill = _build_fill_call()


def kernel(grid, mask, color):
    g2 = grid.reshape(_B * 2, _CW)
    m2 = mask.reshape(_B, _HW)
    color = jnp.asarray(color)
    valid = (color >= 0) & (color < _C)
    safe = jnp.clip(color, 0, _C - 1)
    fill = (jnp.arange(_C) == safe).astype(jnp.float32)
    fill16 = jnp.broadcast_to(fill[:, None], (_C, _L)).reshape(_C * _L)
    thr = jnp.where(valid, jnp.float32(0.5), jnp.float32(3.0e38))
    thr16 = jnp.broadcast_to(thr, (_L,))
    out = _sc_fill(g2, m2, fill16, thr16)
    return out.reshape(grid.shape)


# speedup vs baseline: 9.5454x; 9.5454x over previous
import jax
import jax.numpy as jnp
from jax.experimental import pallas as pl
from jax.experimental.pallas import tpu as pltpu

_B, _C, _HW = 1024, 10, 4096
_P = 128


def _fill_body(g_ref, m_ref, par_ref, o_ref):
    pred = m_ref[...] > par_ref[_C]
    for c in range(_C):
        o_ref[c] = jnp.where(pred, par_ref[c], g_ref[c])


def _tc_fill(gT, mT, params):
    return pl.pallas_call(
        _fill_body,
        grid=(_HW // _P,),
        in_specs=[
            pl.BlockSpec((_C, _P, _B), lambda j: (0, j, 0)),
            pl.BlockSpec((_P, _B), lambda j: (j, 0)),
            pl.BlockSpec(memory_space=pltpu.SMEM),
        ],
        out_specs=pl.BlockSpec((_C, _P, _B), lambda j: (0, j, 0)),
        out_shape=jax.ShapeDtypeStruct((_C, _HW, _B), jnp.float32),
        compiler_params=pltpu.CompilerParams(
            dimension_semantics=("arbitrary",)),
    )(gT, mT, params)


def kernel(grid, mask, color):
    gT = jnp.transpose(grid, (1, 2, 3, 0)).reshape(_C, _HW, _B)
    mT = jnp.transpose(mask, (1, 2, 3, 0)).reshape(_HW, _B)
    color = jnp.asarray(color)
    valid = (color >= 0) & (color < _C)
    safe = jnp.clip(color, 0, _C - 1)
    fill = (jnp.arange(_C) == safe).astype(jnp.float32)
    thr = jnp.where(valid, jnp.float32(0.5), jnp.float32(3.0e38))
    params = jnp.concatenate([fill, thr[None]])
    out = _tc_fill(gT, mT, params)
    return jnp.transpose(out.reshape(_C, 64, 64, _B), (3, 0, 1, 2))
